# baseline (speedup 1.0000x reference)
import functools

import jax
import jax.numpy as jnp
from jax import lax
from jax.experimental import pallas as pl
from jax.experimental.pallas import tpu as pltpu
from jax.experimental.pallas import tpu_sc as plsc

B, S, D, E = 2, 2048, 1024, 8
TOTAL = B * S
CAPACITY = int(TOTAL / E * 1.25)
BLK = 128
NB = 24
PADROWS = NB * BLK
YROWS = PADROWS + BLK
CB = 512
JC = 768

NC, NS = 2, 16
NW = NC * NS


def _router_body(x_ref, wg_ref, bg_ref,
                 srow_ref, inv_ref, be_ref, loss_ref,
                 ohf_ref, ranks_ref):
    xx = x_ref[...]
    logits = jnp.dot(xx, wg_ref[...],
                     preferred_element_type=jnp.float32) + bg_ref[...]
    mx = jnp.max(logits, axis=1, keepdims=True)
    ex = jnp.exp(logits - mx)
    sm = ex / jnp.sum(ex, axis=1, keepdims=True)
    probs_sum = jnp.sum(sm, axis=0, keepdims=True)

    eids = lax.broadcasted_iota(jnp.int32, (TOTAL, E), 1)
    idx2d = jnp.min(jnp.where(logits == mx, eids, E), axis=1, keepdims=True)
    oh = idx2d == eids
    ohf = oh.astype(jnp.float32)
    hist0 = jnp.sum(ohf[:S], axis=0, keepdims=True)
    hist1 = jnp.sum(ohf[S:], axis=0, keepdims=True)

    tri = (lax.broadcasted_iota(jnp.int32, (CB, CB), 0) >=
           lax.broadcasted_iota(jnp.int32, (CB, CB), 1)).astype(jnp.float32)
    ohf_ref[...] = ohf

    def _cum_step(i, carry):
        blk = ohf_ref[pl.ds(i * CB, CB), :]
        csum = jnp.dot(tri, blk, preferred_element_type=jnp.float32) + carry
        ranks_ref[pl.ds(i * CB, CB), :] = csum
        return csum[CB - 1:CB, :]

    lax.fori_loop(0, TOTAL // CB, _cum_step, jnp.zeros((1, E), jnp.float32))
    inc = ranks_ref[...]

    rank_own = jnp.sum(inc * ohf, axis=1, keepdims=True)
    kept = rank_own <= float(CAPACITY)
    contrib = jnp.where(kept, idx2d, -1)
    tm = jnp.maximum(contrib[:S], contrib[S:])

    e_row = lax.broadcasted_iota(jnp.int32, (1, E), 1)
    sp0 = jnp.max(jnp.where(hist0 > 0, e_row, -1))
    sp1 = jnp.max(jnp.where((hist1 > 0) & (hist0 < CAPACITY), e_row, -1))
    sidx = lax.broadcasted_iota(jnp.int32, (S, 1), 0)
    tm = jnp.where(sidx == 0, jnp.maximum(tm, sp0), tm)
    tm = jnp.where(sidx == 1, jnp.maximum(tm, sp1), tm)

    oh_tm = tm == lax.broadcasted_iota(jnp.int32, (S, E), 1)
    ohtf = oh_tm.astype(jnp.float32)
    cnt = jnp.sum(ohtf, axis=0, keepdims=True).astype(jnp.int32)
    pc = ((cnt + (BLK - 1)) // BLK) * BLK
    triu = (lax.broadcasted_iota(jnp.int32, (E, E), 0) <
            lax.broadcasted_iota(jnp.int32, (E, E), 1)).astype(jnp.float32)
    off = jnp.dot(pc.astype(jnp.float32), triu,
                  preferred_element_type=jnp.float32).astype(jnp.int32)

    ohf_ref[pl.ds(0, S), :] = ohtf

    def _cum_step2(i, carry):
        blk = ohf_ref[pl.ds(i * CB, CB), :]
        csum = jnp.dot(tri, blk, preferred_element_type=jnp.float32) + carry
        ranks_ref[pl.ds(i * CB, CB), :] = csum
        return csum[CB - 1:CB, :]

    lax.fori_loop(0, S // CB, _cum_step2, jnp.zeros((1, E), jnp.float32))
    rk = ranks_ref[pl.ds(0, S), :]
    rank_tm = jnp.sum(rk * ohtf, axis=1, keepdims=True).astype(jnp.int32)
    off_own = jnp.sum(jnp.where(oh_tm, jnp.broadcast_to(off, (S, E)), 0),
                      axis=1, keepdims=True)
    valid = tm >= 0
    pos = off_own + rank_tm - 1

    inv_ref[pl.ds(0, S), :] = jnp.where(valid, pos, PADROWS)
    inv_ref[pl.ds(S, S), :] = jnp.full((S, 1), PADROWS, jnp.int32)

    svals = lax.broadcasted_iota(jnp.int32, (S, 1), 0)
    posv = jnp.where(valid, pos, -1)
    for c in range(PADROWS // JC):
        jj = c * JC + lax.broadcasted_iota(jnp.int32, (S, JC), 1)
        hit = posv == jj
        vals = jnp.sum(jnp.where(hit, jnp.broadcast_to(svals, (S, JC)), 0),
                       axis=0, keepdims=True)
        srow_ref[:, pl.ds(c * JC, JC)] = vals

    bj = BLK * lax.broadcasted_iota(jnp.int32, (NB, 1), 0)
    offb = jnp.broadcast_to(off, (NB, E))
    pcb = jnp.broadcast_to(pc, (NB, E))
    inb = (bj >= offb) & (bj < offb + pcb)
    be = jnp.sum(jnp.where(inb, lax.broadcasted_iota(jnp.int32, (NB, E), 1), 0),
                 axis=1, keepdims=True)
    be_ref[...] = be

    dens = (hist0 + hist1) * (1.0 / TOTAL)
    probs = probs_sum * (1.0 / TOTAL)
    loss_ref[...] = jnp.sum(probs * dens, axis=1, keepdims=True) * float(E)


def _run_router(xf, Wg, bg2):
    return pl.pallas_call(
        _router_body,
        out_shape=[
            jax.ShapeDtypeStruct((1, PADROWS), jnp.int32),
            jax.ShapeDtypeStruct((TOTAL, 1), jnp.int32),
            jax.ShapeDtypeStruct((NB, 1), jnp.int32),
            jax.ShapeDtypeStruct((1, 1), jnp.float32),
        ],
        scratch_shapes=[
            pltpu.VMEM((TOTAL, E), jnp.float32),
            pltpu.VMEM((TOTAL, E), jnp.float32),
        ],
    )(xf, Wg, bg2)


def _ffn_body(be_ref, x_ref, w1_ref, b1_ref, w2_ref, b2_ref, y_ref):
    i = pl.program_id(0)
    xb = x_ref[...].astype(jnp.bfloat16)
    w1b = w1_ref[0].astype(jnp.bfloat16)
    h = jnp.maximum(
        jnp.dot(xb, w1b, preferred_element_type=jnp.float32) + b1_ref[0], 0.0)
    w2b = w2_ref[0].astype(jnp.bfloat16)
    y = jnp.dot(h.astype(jnp.bfloat16), w2b,
                preferred_element_type=jnp.float32) + b2_ref[0]
    y_ref[...] = jnp.where(i == NB, 0.0, y)


def _run_ffn(be_flat, x_sorted, W1, b1, W2, b2):
    last = NB - 1
    grid_spec = pltpu.PrefetchScalarGridSpec(
        num_scalar_prefetch=1,
        grid=(NB + 1,),
        in_specs=[
            pl.BlockSpec((BLK, D), lambda i, be: (jnp.minimum(i, last), 0)),
            pl.BlockSpec((1, D, D), lambda i, be: (be[jnp.minimum(i, last)], 0, 0)),
            pl.BlockSpec((1, 1, D), lambda i, be: (be[jnp.minimum(i, last)], 0, 0)),
            pl.BlockSpec((1, D, D), lambda i, be: (be[jnp.minimum(i, last)], 0, 0)),
            pl.BlockSpec((1, 1, D), lambda i, be: (be[jnp.minimum(i, last)], 0, 0)),
        ],
        out_specs=pl.BlockSpec((BLK, D), lambda i, be: (i, 0)),
    )
    return pl.pallas_call(
        _ffn_body,
        grid_spec=grid_spec,
        out_shape=jax.ShapeDtypeStruct((YROWS, D), jnp.float32),
    )(be_flat, x_sorted, W1, b1.reshape(E, 1, D), W2, b2.reshape(E, 1, D))


def _make_sc_gather(n_rows_table, n_out, n_chunks):
    b_per_w = n_out // NW
    c = b_per_w // n_chunks
    mesh = plsc.VectorSubcoreMesh(core_axis_name="c", subcore_axis_name="s",
                                  num_cores=NC, num_subcores=NS)

    def body(table_hbm, idx_hbm, out_hbm, idx_v, rows_v, sem):
        wid = lax.axis_index("s") * NC + lax.axis_index("c")
        base = wid * b_per_w
        pltpu.sync_copy(idx_hbm.at[pl.ds(base, b_per_w)], idx_v)
        for k in range(n_chunks):
            off = base + k * c
            pltpu.async_copy(
                table_hbm.at[idx_v.at[pl.ds(k * c, c)]], rows_v, sem).wait()
            pltpu.sync_copy(rows_v, out_hbm.at[pl.ds(off, c)])

    return pl.kernel(
        body,
        out_type=jax.ShapeDtypeStruct((n_out, D), jnp.float32),
        mesh=mesh,
        scratch_types=[
            pltpu.VMEM((b_per_w,), jnp.int32),
            pltpu.VMEM((c, D), jnp.float32),
            pltpu.SemaphoreType.DMA,
        ],
    )


def _make_sc_combine():
    bw = S // NW
    mesh = plsc.VectorSubcoreMesh(core_axis_name="c", subcore_axis_name="s",
                                  num_cores=NC, num_subcores=NS)

    def body(table_hbm, idx_hbm, zeros_hbm, out_hbm, idx_v, rows_v, zbuf,
             gsem, zsem):
        wid = lax.axis_index("s") * NC + lax.axis_index("c")
        base = wid * bw
        zh = pltpu.async_copy(zeros_hbm, zbuf, zsem)
        pltpu.sync_copy(idx_hbm.at[pl.ds(base, bw)], idx_v)
        pltpu.async_copy(table_hbm.at[idx_v], rows_v, gsem).wait()
        pltpu.sync_copy(rows_v, out_hbm.at[pl.ds(base, bw)])
        zh.wait()
        pltpu.sync_copy(zbuf, out_hbm.at[pl.ds(S + base, bw // 2)])
        pltpu.sync_copy(zbuf, out_hbm.at[pl.ds(S + base + bw // 2, bw // 2)])

    return pl.kernel(
        body,
        out_type=jax.ShapeDtypeStruct((TOTAL, D), jnp.float32),
        mesh=mesh,
        scratch_types=[
            pltpu.VMEM((bw,), jnp.int32),
            pltpu.VMEM((bw, D), jnp.float32),
            pltpu.VMEM((bw // 2, D), jnp.float32),
            pltpu.SemaphoreType.DMA,
            pltpu.SemaphoreType.DMA,
        ],
    )


@jax.jit
def kernel(x, Wg, bg, W1, b1, W2, b2):
    xf = x.reshape(TOTAL, D)
    srow, inv, be, loss = _run_router(xf, Wg, bg.reshape(1, E))
    x_sorted = _make_sc_gather(TOTAL, PADROWS, 1)(xf, srow.reshape(PADROWS))
    y_ext = _run_ffn(be.reshape(NB), x_sorted, W1, b1, W2, b2)
    zeros32 = jnp.zeros((S // NW // 2, D), jnp.float32)
    out = _make_sc_combine()(y_ext, inv.reshape(TOTAL)[:S], zeros32)
    return out.reshape(B, S, D), loss[0, 0]


# speedup vs baseline: 1.4686x; 1.4686x over previous
import functools

import jax
import jax.numpy as jnp
from jax import lax
from jax.experimental import pallas as pl
from jax.experimental.pallas import tpu as pltpu
from jax.experimental.pallas import tpu_sc as plsc

B, S, D, E = 2, 2048, 1024, 8
TOTAL = B * S
CAPACITY = int(TOTAL / E * 1.25)
BLK = 128
NB = 24
PADROWS = NB * BLK
YROWS = PADROWS + BLK
CB = 512
JC = 768

NC, NS = 2, 16
NW = NC * NS


def _router_body(x_ref, wg_ref, bg_ref,
                 srow_ref, inv_ref, be_ref, loss_ref,
                 ohf_ref, ranks_ref):
    xx = x_ref[...]
    logits = jnp.dot(xx, wg_ref[...],
                     preferred_element_type=jnp.float32) + bg_ref[...]
    mx = jnp.max(logits, axis=1, keepdims=True)
    ex = jnp.exp(logits - mx)
    sm = ex / jnp.sum(ex, axis=1, keepdims=True)
    probs_sum = jnp.sum(sm, axis=0, keepdims=True)

    eids = lax.broadcasted_iota(jnp.int32, (TOTAL, E), 1)
    idx2d = jnp.min(jnp.where(logits == mx, eids, E), axis=1, keepdims=True)
    oh = idx2d == eids
    ohf = oh.astype(jnp.float32)
    hist0 = jnp.sum(ohf[:S], axis=0, keepdims=True)
    hist1 = jnp.sum(ohf[S:], axis=0, keepdims=True)

    tri = (lax.broadcasted_iota(jnp.int32, (CB, CB), 0) >=
           lax.broadcasted_iota(jnp.int32, (CB, CB), 1)).astype(jnp.float32)
    ohf_ref[...] = ohf

    def _cum_step(i, carry):
        blk = ohf_ref[pl.ds(i * CB, CB), :]
        csum = jnp.dot(tri, blk, preferred_element_type=jnp.float32) + carry
        ranks_ref[pl.ds(i * CB, CB), :] = csum
        return csum[CB - 1:CB, :]

    lax.fori_loop(0, TOTAL // CB, _cum_step, jnp.zeros((1, E), jnp.float32))
    inc = ranks_ref[...]

    rank_own = jnp.sum(inc * ohf, axis=1, keepdims=True)
    kept = rank_own <= float(CAPACITY)
    contrib = jnp.where(kept, idx2d, -1)
    tm = jnp.maximum(contrib[:S], contrib[S:])

    e_row = lax.broadcasted_iota(jnp.int32, (1, E), 1)
    sp0 = jnp.max(jnp.where(hist0 > 0, e_row, -1))
    sp1 = jnp.max(jnp.where((hist1 > 0) & (hist0 < CAPACITY), e_row, -1))
    sidx = lax.broadcasted_iota(jnp.int32, (S, 1), 0)
    tm = jnp.where(sidx == 0, jnp.maximum(tm, sp0), tm)
    tm = jnp.where(sidx == 1, jnp.maximum(tm, sp1), tm)

    oh_tm = tm == lax.broadcasted_iota(jnp.int32, (S, E), 1)
    ohtf = oh_tm.astype(jnp.float32)
    cnt = jnp.sum(ohtf, axis=0, keepdims=True).astype(jnp.int32)
    pc = ((cnt + (BLK - 1)) // BLK) * BLK
    triu = (lax.broadcasted_iota(jnp.int32, (E, E), 0) <
            lax.broadcasted_iota(jnp.int32, (E, E), 1)).astype(jnp.float32)
    off = jnp.dot(pc.astype(jnp.float32), triu,
                  preferred_element_type=jnp.float32).astype(jnp.int32)

    ohf_ref[pl.ds(0, S), :] = ohtf

    def _cum_step2(i, carry):
        blk = ohf_ref[pl.ds(i * CB, CB), :]
        csum = jnp.dot(tri, blk, preferred_element_type=jnp.float32) + carry
        ranks_ref[pl.ds(i * CB, CB), :] = csum
        return csum[CB - 1:CB, :]

    lax.fori_loop(0, S // CB, _cum_step2, jnp.zeros((1, E), jnp.float32))
    rk = ranks_ref[pl.ds(0, S), :]
    rank_tm = jnp.sum(rk * ohtf, axis=1, keepdims=True).astype(jnp.int32)
    off_own = jnp.sum(jnp.where(oh_tm, jnp.broadcast_to(off, (S, E)), 0),
                      axis=1, keepdims=True)
    valid = tm >= 0
    pos = off_own + rank_tm - 1

    inv_ref[pl.ds(0, S), :] = jnp.where(valid, pos, PADROWS)
    inv_ref[pl.ds(S, S), :] = jnp.full((S, 1), PADROWS, jnp.int32)

    svals = lax.broadcasted_iota(jnp.int32, (S, 1), 0)
    posv = jnp.where(valid, pos, -1)
    for c in range(PADROWS // JC):
        jj = c * JC + lax.broadcasted_iota(jnp.int32, (S, JC), 1)
        hit = posv == jj
        vals = jnp.sum(jnp.where(hit, jnp.broadcast_to(svals, (S, JC)), 0),
                       axis=0, keepdims=True)
        occ = jnp.sum(hit.astype(jnp.int32), axis=0, keepdims=True)
        jrow = c * JC + lax.broadcasted_iota(jnp.int32, (1, JC), 1)
        srow_ref[:, pl.ds(c * JC, JC)] = jnp.where(occ > 0, vals, jrow)

    bj = BLK * lax.broadcasted_iota(jnp.int32, (NB, 1), 0)
    offb = jnp.broadcast_to(off, (NB, E))
    pcb = jnp.broadcast_to(pc, (NB, E))
    inb = (bj >= offb) & (bj < offb + pcb)
    be = jnp.sum(jnp.where(inb, lax.broadcasted_iota(jnp.int32, (NB, E), 1), 0),
                 axis=1, keepdims=True)
    be_ref[...] = be

    dens = (hist0 + hist1) * (1.0 / TOTAL)
    probs = probs_sum * (1.0 / TOTAL)
    loss_ref[...] = jnp.sum(probs * dens, axis=1, keepdims=True) * float(E)


def _run_router(xf, Wg, bg2):
    return pl.pallas_call(
        _router_body,
        out_shape=[
            jax.ShapeDtypeStruct((1, PADROWS), jnp.int32),
            jax.ShapeDtypeStruct((TOTAL, 1), jnp.int32),
            jax.ShapeDtypeStruct((NB, 1), jnp.int32),
            jax.ShapeDtypeStruct((1, 1), jnp.float32),
        ],
        scratch_shapes=[
            pltpu.VMEM((TOTAL, E), jnp.float32),
            pltpu.VMEM((TOTAL, E), jnp.float32),
        ],
    )(xf, Wg, bg2)


def _ffn_body(be_ref, x_ref, w1_ref, b1_ref, w2_ref, b2_ref, y_ref):
    i = pl.program_id(0)
    h = jnp.maximum(
        jnp.dot(x_ref[...], w1_ref[0], preferred_element_type=jnp.float32)
        + b1_ref[0], 0.0)
    y = jnp.dot(h, w2_ref[0], preferred_element_type=jnp.float32) + b2_ref[0]
    y_ref[...] = jnp.where(i == NB, 0.0, y)


def _run_ffn(be_flat, x_sorted, W1, b1, W2, b2):
    last = NB - 1
    grid_spec = pltpu.PrefetchScalarGridSpec(
        num_scalar_prefetch=1,
        grid=(NB + 1,),
        in_specs=[
            pl.BlockSpec((BLK, D), lambda i, be: (jnp.minimum(i, last), 0)),
            pl.BlockSpec((1, D, D), lambda i, be: (be[jnp.minimum(i, last)], 0, 0)),
            pl.BlockSpec((1, 1, D), lambda i, be: (be[jnp.minimum(i, last)], 0, 0)),
            pl.BlockSpec((1, D, D), lambda i, be: (be[jnp.minimum(i, last)], 0, 0)),
            pl.BlockSpec((1, 1, D), lambda i, be: (be[jnp.minimum(i, last)], 0, 0)),
        ],
        out_specs=pl.BlockSpec((BLK, D), lambda i, be: (i, 0)),
    )
    return pl.pallas_call(
        _ffn_body,
        grid_spec=grid_spec,
        out_shape=jax.ShapeDtypeStruct((YROWS, D), jnp.float32),
    )(be_flat, x_sorted, W1, b1.reshape(E, 1, D), W2, b2.reshape(E, 1, D))


def _make_sc_gather(n_rows_table, n_out, n_chunks):
    b_per_w = n_out // NW
    c = b_per_w // n_chunks
    mesh = plsc.VectorSubcoreMesh(core_axis_name="c", subcore_axis_name="s",
                                  num_cores=NC, num_subcores=NS)

    def body(table_hbm, idx_hbm, out_hbm, idx_v, rows_v, sem):
        wid = lax.axis_index("s") * NC + lax.axis_index("c")
        base = wid * b_per_w
        pltpu.sync_copy(idx_hbm.at[pl.ds(base, b_per_w)], idx_v)
        for k in range(n_chunks):
            off = base + k * c
            pltpu.async_copy(
                table_hbm.at[idx_v.at[pl.ds(k * c, c)]], rows_v, sem).wait()
            pltpu.sync_copy(rows_v, out_hbm.at[pl.ds(off, c)])

    return pl.kernel(
        body,
        out_type=jax.ShapeDtypeStruct((n_out, D), jnp.float32),
        mesh=mesh,
        scratch_types=[
            pltpu.VMEM((b_per_w,), jnp.int32),
            pltpu.VMEM((c, D), jnp.float32),
            pltpu.SemaphoreType.DMA,
        ],
    )


def _make_sc_combine():
    bw = S // NW
    mesh = plsc.VectorSubcoreMesh(core_axis_name="c", subcore_axis_name="s",
                                  num_cores=NC, num_subcores=NS)

    def body(table_hbm, idx_hbm, zeros_hbm, out_hbm, idx_v, rows_v, zbuf,
             gsem, zsem):
        wid = lax.axis_index("s") * NC + lax.axis_index("c")
        base = wid * bw
        zh = pltpu.async_copy(zeros_hbm, zbuf, zsem)
        pltpu.sync_copy(idx_hbm.at[pl.ds(base, bw)], idx_v)
        pltpu.async_copy(table_hbm.at[idx_v], rows_v, gsem).wait()
        pltpu.sync_copy(rows_v, out_hbm.at[pl.ds(base, bw)])
        zh.wait()
        pltpu.sync_copy(zbuf, out_hbm.at[pl.ds(S + base, bw // 2)])
        pltpu.sync_copy(zbuf, out_hbm.at[pl.ds(S + base + bw // 2, bw // 2)])

    return pl.kernel(
        body,
        out_type=jax.ShapeDtypeStruct((TOTAL, D), jnp.float32),
        mesh=mesh,
        scratch_types=[
            pltpu.VMEM((bw,), jnp.int32),
            pltpu.VMEM((bw, D), jnp.float32),
            pltpu.VMEM((bw // 2, D), jnp.float32),
            pltpu.SemaphoreType.DMA,
            pltpu.SemaphoreType.DMA,
        ],
    )


@jax.jit
def kernel(x, Wg, bg, W1, b1, W2, b2):
    xf = x.reshape(TOTAL, D)
    srow, inv, be, loss = _run_router(xf, Wg, bg.reshape(1, E))
    x_sorted = _make_sc_gather(TOTAL, PADROWS, 1)(xf, srow.reshape(PADROWS))
    y_ext = _run_ffn(be.reshape(NB), x_sorted, W1, b1, W2, b2)
    zeros32 = jnp.zeros((S // NW // 2, D), jnp.float32)
    out = _make_sc_combine()(y_ext, inv.reshape(TOTAL)[:S], zeros32)
    return out.reshape(B, S, D), loss[0, 0]


# speedup vs baseline: 1.4745x; 1.0040x over previous
import functools

import jax
import jax.numpy as jnp
from jax import lax
from jax.experimental import pallas as pl
from jax.experimental.pallas import tpu as pltpu
from jax.experimental.pallas import tpu_sc as plsc

B, S, D, E = 2, 2048, 1024, 8
TOTAL = B * S
CAPACITY = int(TOTAL / E * 1.25)
BLK = 128
NB = 24
PADROWS = NB * BLK
YROWS = PADROWS + BLK
CB = 512
JC = 768

NC, NS = 2, 16
NW = NC * NS


def _router_body(x_ref, wg_ref, bg_ref,
                 srow_ref, inv_ref, be_ref, loss_ref,
                 ohf_ref, ranks_ref):
    xx = x_ref[...]
    logits = jnp.dot(xx, wg_ref[...],
                     preferred_element_type=jnp.float32) + bg_ref[...]
    mx = jnp.max(logits, axis=1, keepdims=True)
    ex = jnp.exp(logits - mx)
    sm = ex / jnp.sum(ex, axis=1, keepdims=True)
    probs_sum = jnp.sum(sm, axis=0, keepdims=True)

    eids = lax.broadcasted_iota(jnp.int32, (TOTAL, E), 1)
    idx2d = jnp.min(jnp.where(logits == mx, eids, E), axis=1, keepdims=True)
    oh = idx2d == eids
    ohf = oh.astype(jnp.float32)
    hist0 = jnp.sum(ohf[:S], axis=0, keepdims=True)
    hist1 = jnp.sum(ohf[S:], axis=0, keepdims=True)

    tri = (lax.broadcasted_iota(jnp.int32, (CB, CB), 0) >=
           lax.broadcasted_iota(jnp.int32, (CB, CB), 1)).astype(jnp.float32)
    ohf_ref[...] = ohf

    def _cum_step(i, carry):
        blk = ohf_ref[pl.ds(i * CB, CB), :]
        csum = jnp.dot(tri, blk, preferred_element_type=jnp.float32) + carry
        ranks_ref[pl.ds(i * CB, CB), :] = csum
        return csum[CB - 1:CB, :]

    lax.fori_loop(0, TOTAL // CB, _cum_step, jnp.zeros((1, E), jnp.float32))
    inc = ranks_ref[...]

    rank_own = jnp.sum(inc * ohf, axis=1, keepdims=True)
    kept = rank_own <= float(CAPACITY)
    contrib = jnp.where(kept, idx2d, -1)
    tm = jnp.maximum(contrib[:S], contrib[S:])

    e_row = lax.broadcasted_iota(jnp.int32, (1, E), 1)
    sp0 = jnp.max(jnp.where(hist0 > 0, e_row, -1))
    sp1 = jnp.max(jnp.where((hist1 > 0) & (hist0 < CAPACITY), e_row, -1))
    sidx = lax.broadcasted_iota(jnp.int32, (S, 1), 0)
    tm = jnp.where(sidx == 0, jnp.maximum(tm, sp0), tm)
    tm = jnp.where(sidx == 1, jnp.maximum(tm, sp1), tm)

    oh_tm = tm == lax.broadcasted_iota(jnp.int32, (S, E), 1)
    ohtf = oh_tm.astype(jnp.float32)
    cnt = jnp.sum(ohtf, axis=0, keepdims=True).astype(jnp.int32)
    pc = ((cnt + (BLK - 1)) // BLK) * BLK
    triu = (lax.broadcasted_iota(jnp.int32, (E, E), 0) <
            lax.broadcasted_iota(jnp.int32, (E, E), 1)).astype(jnp.float32)
    off = jnp.dot(pc.astype(jnp.float32), triu,
                  preferred_element_type=jnp.float32).astype(jnp.int32)

    ohf_ref[pl.ds(0, S), :] = ohtf

    def _cum_step2(i, carry):
        blk = ohf_ref[pl.ds(i * CB, CB), :]
        csum = jnp.dot(tri, blk, preferred_element_type=jnp.float32) + carry
        ranks_ref[pl.ds(i * CB, CB), :] = csum
        return csum[CB - 1:CB, :]

    lax.fori_loop(0, S // CB, _cum_step2, jnp.zeros((1, E), jnp.float32))
    rk = ranks_ref[pl.ds(0, S), :]
    rank_tm = jnp.sum(rk * ohtf, axis=1, keepdims=True).astype(jnp.int32)
    off_own = jnp.sum(jnp.where(oh_tm, jnp.broadcast_to(off, (S, E)), 0),
                      axis=1, keepdims=True)
    valid = tm >= 0
    pos = off_own + rank_tm - 1

    inv_ref[...] = jnp.where(valid, pos, PADROWS)

    svals = lax.broadcasted_iota(jnp.int32, (S, 1), 0)
    posv = jnp.where(valid, pos, -1)
    for c in range(PADROWS // JC):
        jj = c * JC + lax.broadcasted_iota(jnp.int32, (S, JC), 1)
        hit = posv == jj
        vals = jnp.sum(jnp.where(hit, jnp.broadcast_to(svals, (S, JC)), 0),
                       axis=0, keepdims=True)
        occ = jnp.sum(hit.astype(jnp.int32), axis=0, keepdims=True)
        jrow = c * JC + lax.broadcasted_iota(jnp.int32, (1, JC), 1)
        srow_ref[:, pl.ds(c * JC, JC)] = jnp.where(occ > 0, vals, jrow)

    bj = BLK * lax.broadcasted_iota(jnp.int32, (NB, 1), 0)
    offb = jnp.broadcast_to(off, (NB, E))
    pcb = jnp.broadcast_to(pc, (NB, E))
    inb = (bj >= offb) & (bj < offb + pcb)
    be = jnp.sum(jnp.where(inb, lax.broadcasted_iota(jnp.int32, (NB, E), 1), 0),
                 axis=1, keepdims=True)
    be_ref[...] = be

    dens = (hist0 + hist1) * (1.0 / TOTAL)
    probs = probs_sum * (1.0 / TOTAL)
    loss_ref[...] = jnp.sum(probs * dens, axis=1, keepdims=True) * float(E)


def _run_router(xf, Wg, bg2):
    return pl.pallas_call(
        _router_body,
        out_shape=[
            jax.ShapeDtypeStruct((1, PADROWS), jnp.int32),
            jax.ShapeDtypeStruct((S, 1), jnp.int32),
            jax.ShapeDtypeStruct((NB, 1), jnp.int32),
            jax.ShapeDtypeStruct((1, 1), jnp.float32),
        ],
        scratch_shapes=[
            pltpu.VMEM((TOTAL, E), jnp.float32),
            pltpu.VMEM((TOTAL, E), jnp.float32),
        ],
    )(xf, Wg, bg2)


def _ffn_body(be_ref, x_ref, w1_ref, b1_ref, w2_ref, b2_ref, y_ref):
    i = pl.program_id(0)
    h = jnp.maximum(
        jnp.dot(x_ref[...], w1_ref[0], preferred_element_type=jnp.float32)
        + b1_ref[0], 0.0)
    y = jnp.dot(h, w2_ref[0], preferred_element_type=jnp.float32) + b2_ref[0]
    y_ref[...] = jnp.where(i == NB, 0.0, y)


def _run_ffn(be_flat, x_sorted, W1, b1, W2, b2):
    last = NB - 1
    grid_spec = pltpu.PrefetchScalarGridSpec(
        num_scalar_prefetch=1,
        grid=(NB + 1,),
        in_specs=[
            pl.BlockSpec((BLK, D), lambda i, be: (jnp.minimum(i, last), 0)),
            pl.BlockSpec((1, D, D), lambda i, be: (be[jnp.minimum(i, last)], 0, 0)),
            pl.BlockSpec((1, 1, D), lambda i, be: (be[jnp.minimum(i, last)], 0, 0)),
            pl.BlockSpec((1, D, D), lambda i, be: (be[jnp.minimum(i, last)], 0, 0)),
            pl.BlockSpec((1, 1, D), lambda i, be: (be[jnp.minimum(i, last)], 0, 0)),
        ],
        out_specs=pl.BlockSpec((BLK, D), lambda i, be: (i, 0)),
    )
    return pl.pallas_call(
        _ffn_body,
        grid_spec=grid_spec,
        out_shape=jax.ShapeDtypeStruct((YROWS, D), jnp.float32),
    )(be_flat, x_sorted, W1, b1.reshape(E, 1, D), W2, b2.reshape(E, 1, D))


def _make_sc_gather(n_rows_table, n_out, n_chunks):
    b_per_w = n_out // NW
    c = b_per_w // n_chunks
    mesh = plsc.VectorSubcoreMesh(core_axis_name="c", subcore_axis_name="s",
                                  num_cores=NC, num_subcores=NS)

    def body(table_hbm, idx_hbm, out_hbm, idx_v, rows_v, sem):
        wid = lax.axis_index("s") * NC + lax.axis_index("c")
        base = wid * b_per_w
        pltpu.sync_copy(idx_hbm.at[pl.ds(base, b_per_w)], idx_v)
        for k in range(n_chunks):
            off = base + k * c
            pltpu.async_copy(
                table_hbm.at[idx_v.at[pl.ds(k * c, c)]], rows_v, sem).wait()
            pltpu.sync_copy(rows_v, out_hbm.at[pl.ds(off, c)])

    return pl.kernel(
        body,
        out_type=jax.ShapeDtypeStruct((n_out, D), jnp.float32),
        mesh=mesh,
        scratch_types=[
            pltpu.VMEM((b_per_w,), jnp.int32),
            pltpu.VMEM((c, D), jnp.float32),
            pltpu.SemaphoreType.DMA,
        ],
    )


def _make_sc_combine():
    bw = S // NW
    mesh = plsc.VectorSubcoreMesh(core_axis_name="c", subcore_axis_name="s",
                                  num_cores=NC, num_subcores=NS)

    def body(table_hbm, idx_hbm, zeros_hbm, out_hbm, idx_v, rows_v, zbuf,
             gsem, zsem):
        wid = lax.axis_index("s") * NC + lax.axis_index("c")
        base = wid * bw
        zh = pltpu.async_copy(zeros_hbm, zbuf, zsem)
        pltpu.sync_copy(idx_hbm.at[pl.ds(base, bw)], idx_v)
        pltpu.async_copy(table_hbm.at[idx_v], rows_v, gsem).wait()
        pltpu.sync_copy(rows_v, out_hbm.at[pl.ds(base, bw)])
        zh.wait()
        pltpu.sync_copy(zbuf, out_hbm.at[pl.ds(S + base, bw // 2)])
        pltpu.sync_copy(zbuf, out_hbm.at[pl.ds(S + base + bw // 2, bw // 2)])

    return pl.kernel(
        body,
        out_type=jax.ShapeDtypeStruct((TOTAL, D), jnp.float32),
        mesh=mesh,
        scratch_types=[
            pltpu.VMEM((bw,), jnp.int32),
            pltpu.VMEM((bw, D), jnp.float32),
            pltpu.VMEM((bw // 2, D), jnp.float32),
            pltpu.SemaphoreType.DMA,
            pltpu.SemaphoreType.DMA,
        ],
    )


@jax.jit
def kernel(x, Wg, bg, W1, b1, W2, b2):
    xf = x.reshape(TOTAL, D)
    srow, inv, be, loss = _run_router(xf, Wg, bg.reshape(1, E))
    x_sorted = _make_sc_gather(TOTAL, PADROWS, 1)(xf, srow.reshape(PADROWS))
    y_ext = _run_ffn(be.reshape(NB), x_sorted, W1, b1, W2, b2)
    zeros32 = jnp.zeros((S // NW // 2, D), jnp.float32)
    out = _make_sc_combine()(y_ext, inv.reshape(S), zeros32)
    return out.reshape(B, S, D), loss[0, 0]


# speedup vs baseline: 1.5088x; 1.0233x over previous
import functools

import jax
import jax.numpy as jnp
from jax import lax
from jax.experimental import pallas as pl
from jax.experimental.pallas import tpu as pltpu
from jax.experimental.pallas import tpu_sc as plsc

B, S, D, E = 2, 2048, 1024, 8
TOTAL = B * S
CAPACITY = int(TOTAL / E * 1.25)
BLK = 128
NB = 24
PADROWS = NB * BLK
YROWS = PADROWS + BLK
CB = 512
JC = 768

NC, NS = 2, 16
NW = NC * NS


def _router_body(x_ref, wg_ref, bg_ref,
                 srow_ref, inv_ref, be_ref, loss_ref,
                 ohf_ref, ranks_ref):
    xx = x_ref[...]
    logits = jnp.dot(xx, wg_ref[...],
                     preferred_element_type=jnp.float32) + bg_ref[...]
    mx = jnp.max(logits, axis=1, keepdims=True)
    ex = jnp.exp(logits - mx)
    sm = ex / jnp.sum(ex, axis=1, keepdims=True)
    probs_sum = jnp.sum(sm, axis=0, keepdims=True)

    eids = lax.broadcasted_iota(jnp.int32, (TOTAL, E), 1)
    idx2d = jnp.min(jnp.where(logits == mx, eids, E), axis=1, keepdims=True)
    oh = idx2d == eids
    ohf = oh.astype(jnp.float32)
    hist0 = jnp.sum(ohf[:S], axis=0, keepdims=True)
    hist1 = jnp.sum(ohf[S:], axis=0, keepdims=True)

    tri = (lax.broadcasted_iota(jnp.int32, (CB, CB), 0) >=
           lax.broadcasted_iota(jnp.int32, (CB, CB), 1)).astype(jnp.float32)
    ohf_ref[...] = ohf

    def _cum_step(i, carry):
        blk = ohf_ref[pl.ds(i * CB, CB), :]
        csum = jnp.dot(tri, blk, preferred_element_type=jnp.float32) + carry
        ranks_ref[pl.ds(i * CB, CB), :] = csum
        return csum[CB - 1:CB, :]

    lax.fori_loop(0, TOTAL // CB, _cum_step, jnp.zeros((1, E), jnp.float32))
    inc = ranks_ref[...]

    rank_own = jnp.sum(inc * ohf, axis=1, keepdims=True)
    kept = rank_own <= float(CAPACITY)
    contrib = jnp.where(kept, idx2d, -1)
    tm = jnp.maximum(contrib[:S], contrib[S:])

    e_row = lax.broadcasted_iota(jnp.int32, (1, E), 1)
    sp0 = jnp.max(jnp.where(hist0 > 0, e_row, -1))
    sp1 = jnp.max(jnp.where((hist1 > 0) & (hist0 < CAPACITY), e_row, -1))
    sidx = lax.broadcasted_iota(jnp.int32, (S, 1), 0)
    tm = jnp.where(sidx == 0, jnp.maximum(tm, sp0), tm)
    tm = jnp.where(sidx == 1, jnp.maximum(tm, sp1), tm)

    oh_tm = tm == lax.broadcasted_iota(jnp.int32, (S, E), 1)
    ohtf = oh_tm.astype(jnp.float32)
    cnt = jnp.sum(ohtf, axis=0, keepdims=True).astype(jnp.int32)
    pc = ((cnt + (BLK - 1)) // BLK) * BLK
    triu = (lax.broadcasted_iota(jnp.int32, (E, E), 0) <
            lax.broadcasted_iota(jnp.int32, (E, E), 1)).astype(jnp.float32)
    off = jnp.dot(pc.astype(jnp.float32), triu,
                  preferred_element_type=jnp.float32).astype(jnp.int32)

    ohf_ref[pl.ds(0, S), :] = ohtf

    def _cum_step2(i, carry):
        blk = ohf_ref[pl.ds(i * CB, CB), :]
        csum = jnp.dot(tri, blk, preferred_element_type=jnp.float32) + carry
        ranks_ref[pl.ds(i * CB, CB), :] = csum
        return csum[CB - 1:CB, :]

    lax.fori_loop(0, S // CB, _cum_step2, jnp.zeros((1, E), jnp.float32))
    rk = ranks_ref[pl.ds(0, S), :]
    rank_tm = jnp.sum(rk * ohtf, axis=1, keepdims=True).astype(jnp.int32)
    off_own = jnp.sum(jnp.where(oh_tm, jnp.broadcast_to(off, (S, E)), 0),
                      axis=1, keepdims=True)
    valid = tm >= 0
    pos = off_own + rank_tm - 1

    inv_ref[...] = jnp.where(valid, pos, PADROWS)

    svals = lax.broadcasted_iota(jnp.int32, (S, 1), 0)
    posv = jnp.where(valid, pos, -1)
    for c in range(PADROWS // JC):
        jj = c * JC + lax.broadcasted_iota(jnp.int32, (S, JC), 1)
        hit = posv == jj
        vals = jnp.sum(jnp.where(hit, jnp.broadcast_to(svals, (S, JC)), 0),
                       axis=0, keepdims=True)
        occ = jnp.sum(hit.astype(jnp.int32), axis=0, keepdims=True)
        jrow = c * JC + lax.broadcasted_iota(jnp.int32, (1, JC), 1)
        srow_ref[:, pl.ds(c * JC, JC)] = jnp.where(occ > 0, vals, jrow)

    bj = BLK * lax.broadcasted_iota(jnp.int32, (NB, 1), 0)
    offb = jnp.broadcast_to(off, (NB, E))
    pcb = jnp.broadcast_to(pc, (NB, E))
    inb = (bj >= offb) & (bj < offb + pcb)
    be = jnp.sum(jnp.where(inb, lax.broadcasted_iota(jnp.int32, (NB, E), 1), 0),
                 axis=1, keepdims=True)
    e_last = jnp.max(jnp.where(pc > 0, e_row, 0))
    used = jnp.sum(inb.astype(jnp.int32), axis=1, keepdims=True) > 0
    be_ref[...] = jnp.where(used, be, e_last)

    dens = (hist0 + hist1) * (1.0 / TOTAL)
    probs = probs_sum * (1.0 / TOTAL)
    loss_ref[...] = jnp.sum(probs * dens, axis=1, keepdims=True) * float(E)


def _run_router(xf, Wg, bg2):
    return pl.pallas_call(
        _router_body,
        out_shape=[
            jax.ShapeDtypeStruct((1, PADROWS), jnp.int32),
            jax.ShapeDtypeStruct((S, 1), jnp.int32),
            jax.ShapeDtypeStruct((NB, 1), jnp.int32),
            jax.ShapeDtypeStruct((1, 1), jnp.float32),
        ],
        scratch_shapes=[
            pltpu.VMEM((TOTAL, E), jnp.float32),
            pltpu.VMEM((TOTAL, E), jnp.float32),
        ],
    )(xf, Wg, bg2)


def _ffn_body(be_ref, x_ref, w1_ref, b1_ref, w2_ref, b2_ref, y_ref):
    i = pl.program_id(0)
    h = jnp.maximum(
        jnp.dot(x_ref[...], w1_ref[0], preferred_element_type=jnp.float32)
        + b1_ref[0], 0.0)
    y = jnp.dot(h, w2_ref[0], preferred_element_type=jnp.float32) + b2_ref[0]
    y_ref[...] = jnp.where(i == NB, 0.0, y)


def _run_ffn(be_flat, x_sorted, W1, b1, W2, b2):
    last = NB - 1
    grid_spec = pltpu.PrefetchScalarGridSpec(
        num_scalar_prefetch=1,
        grid=(NB + 1,),
        in_specs=[
            pl.BlockSpec((BLK, D), lambda i, be: (jnp.minimum(i, last), 0)),
            pl.BlockSpec((1, D, D), lambda i, be: (be[jnp.minimum(i, last)], 0, 0)),
            pl.BlockSpec((1, 1, D), lambda i, be: (be[jnp.minimum(i, last)], 0, 0)),
            pl.BlockSpec((1, D, D), lambda i, be: (be[jnp.minimum(i, last)], 0, 0)),
            pl.BlockSpec((1, 1, D), lambda i, be: (be[jnp.minimum(i, last)], 0, 0)),
        ],
        out_specs=pl.BlockSpec((BLK, D), lambda i, be: (i, 0)),
    )
    return pl.pallas_call(
        _ffn_body,
        grid_spec=grid_spec,
        out_shape=jax.ShapeDtypeStruct((YROWS, D), jnp.float32),
    )(be_flat, x_sorted, W1, b1.reshape(E, 1, D), W2, b2.reshape(E, 1, D))


def _make_sc_gather(n_rows_table, n_out, n_chunks):
    b_per_w = n_out // NW
    c = b_per_w // n_chunks
    mesh = plsc.VectorSubcoreMesh(core_axis_name="c", subcore_axis_name="s",
                                  num_cores=NC, num_subcores=NS)

    def body(table_hbm, idx_hbm, out_hbm, idx_v, rows_v, sem):
        wid = lax.axis_index("s") * NC + lax.axis_index("c")
        base = wid * b_per_w
        pltpu.sync_copy(idx_hbm.at[pl.ds(base, b_per_w)], idx_v)
        for k in range(n_chunks):
            off = base + k * c
            pltpu.async_copy(
                table_hbm.at[idx_v.at[pl.ds(k * c, c)]], rows_v, sem).wait()
            pltpu.sync_copy(rows_v, out_hbm.at[pl.ds(off, c)])

    return pl.kernel(
        body,
        out_type=jax.ShapeDtypeStruct((n_out, D), jnp.float32),
        mesh=mesh,
        scratch_types=[
            pltpu.VMEM((b_per_w,), jnp.int32),
            pltpu.VMEM((c, D), jnp.float32),
            pltpu.SemaphoreType.DMA,
        ],
    )


def _make_sc_combine():
    bw = S // NW
    mesh = plsc.VectorSubcoreMesh(core_axis_name="c", subcore_axis_name="s",
                                  num_cores=NC, num_subcores=NS)

    def body(table_hbm, idx_hbm, zeros_hbm, out_hbm, idx_v, rows_v, zbuf,
             gsem, zsem):
        wid = lax.axis_index("s") * NC + lax.axis_index("c")
        base = wid * bw
        zh = pltpu.async_copy(zeros_hbm, zbuf, zsem)
        pltpu.sync_copy(idx_hbm.at[pl.ds(base, bw)], idx_v)
        pltpu.async_copy(table_hbm.at[idx_v], rows_v, gsem).wait()
        pltpu.sync_copy(rows_v, out_hbm.at[pl.ds(base, bw)])
        zh.wait()
        pltpu.sync_copy(zbuf, out_hbm.at[pl.ds(S + base, bw // 2)])
        pltpu.sync_copy(zbuf, out_hbm.at[pl.ds(S + base + bw // 2, bw // 2)])

    return pl.kernel(
        body,
        out_type=jax.ShapeDtypeStruct((TOTAL, D), jnp.float32),
        mesh=mesh,
        scratch_types=[
            pltpu.VMEM((bw,), jnp.int32),
            pltpu.VMEM((bw, D), jnp.float32),
            pltpu.VMEM((bw // 2, D), jnp.float32),
            pltpu.SemaphoreType.DMA,
            pltpu.SemaphoreType.DMA,
        ],
    )


@jax.jit
def kernel(x, Wg, bg, W1, b1, W2, b2):
    xf = x.reshape(TOTAL, D)
    srow, inv, be, loss = _run_router(xf, Wg, bg.reshape(1, E))
    x_sorted = _make_sc_gather(TOTAL, PADROWS, 1)(xf, srow.reshape(PADROWS))
    y_ext = _run_ffn(be.reshape(NB), x_sorted, W1, b1, W2, b2)
    zeros32 = jnp.zeros((S // NW // 2, D), jnp.float32)
    out = _make_sc_combine()(y_ext, inv.reshape(S), zeros32)
    return out.reshape(B, S, D), loss[0, 0]


# speedup vs baseline: 1.5297x; 1.0139x over previous
import functools

import jax
import jax.numpy as jnp
from jax import lax
from jax.experimental import pallas as pl
from jax.experimental.pallas import tpu as pltpu
from jax.experimental.pallas import tpu_sc as plsc

B, S, D, E = 2, 2048, 1024, 8
TOTAL = B * S
CAPACITY = int(TOTAL / E * 1.25)
BLK = 128
NB = 24
PADROWS = NB * BLK
YROWS = PADROWS + BLK
CB = 512
JC = 768

NC, NS = 2, 16
NW = NC * NS


def _router_body(x_ref, wg_ref, bg_ref,
                 srow_ref, inv_ref, be_ref, loss_ref,
                 ohf_ref, ranks_ref):
    xx = x_ref[...]
    logits = jnp.dot(xx, wg_ref[...],
                     preferred_element_type=jnp.float32) + bg_ref[...]
    mx = jnp.max(logits, axis=1, keepdims=True)
    ex = jnp.exp(logits - mx)
    sm = ex / jnp.sum(ex, axis=1, keepdims=True)
    probs_sum = jnp.sum(sm, axis=0, keepdims=True)

    eids = lax.broadcasted_iota(jnp.int32, (TOTAL, E), 1)
    idx2d = jnp.min(jnp.where(logits == mx, eids, E), axis=1, keepdims=True)
    oh = idx2d == eids
    ohf = oh.astype(jnp.float32)
    hist0 = jnp.sum(ohf[:S], axis=0, keepdims=True)
    hist1 = jnp.sum(ohf[S:], axis=0, keepdims=True)

    tri = (lax.broadcasted_iota(jnp.int32, (CB, CB), 0) >=
           lax.broadcasted_iota(jnp.int32, (CB, CB), 1)).astype(jnp.float32)
    ohf_ref[...] = ohf

    def _cum_step(i, carry):
        blk = ohf_ref[pl.ds(i * CB, CB), :]
        csum = jnp.dot(tri, blk, preferred_element_type=jnp.float32) + carry
        ranks_ref[pl.ds(i * CB, CB), :] = csum
        return csum[CB - 1:CB, :]

    lax.fori_loop(0, TOTAL // CB, _cum_step, jnp.zeros((1, E), jnp.float32))
    inc = ranks_ref[...]

    rank_own = jnp.sum(inc * ohf, axis=1, keepdims=True)
    kept = rank_own <= float(CAPACITY)
    contrib = jnp.where(kept, idx2d, -1)
    tm = jnp.maximum(contrib[:S], contrib[S:])

    e_row = lax.broadcasted_iota(jnp.int32, (1, E), 1)
    sp0 = jnp.max(jnp.where(hist0 > 0, e_row, -1))
    sp1 = jnp.max(jnp.where((hist1 > 0) & (hist0 < CAPACITY), e_row, -1))
    sidx = lax.broadcasted_iota(jnp.int32, (S, 1), 0)
    tm = jnp.where(sidx == 0, jnp.maximum(tm, sp0), tm)
    tm = jnp.where(sidx == 1, jnp.maximum(tm, sp1), tm)

    oh_tm = tm == lax.broadcasted_iota(jnp.int32, (S, E), 1)
    ohtf = oh_tm.astype(jnp.float32)
    cnt = jnp.sum(ohtf, axis=0, keepdims=True).astype(jnp.int32)
    pc = ((cnt + (BLK - 1)) // BLK) * BLK
    triu = (lax.broadcasted_iota(jnp.int32, (E, E), 0) <
            lax.broadcasted_iota(jnp.int32, (E, E), 1)).astype(jnp.float32)
    off = jnp.dot(pc.astype(jnp.float32), triu,
                  preferred_element_type=jnp.float32).astype(jnp.int32)

    ohf_ref[pl.ds(0, S), :] = ohtf

    def _cum_step2(i, carry):
        blk = ohf_ref[pl.ds(i * CB, CB), :]
        csum = jnp.dot(tri, blk, preferred_element_type=jnp.float32) + carry
        ranks_ref[pl.ds(i * CB, CB), :] = csum
        return csum[CB - 1:CB, :]

    lax.fori_loop(0, S // CB, _cum_step2, jnp.zeros((1, E), jnp.float32))
    rk = ranks_ref[pl.ds(0, S), :]
    rank_tm = jnp.sum(rk * ohtf, axis=1, keepdims=True).astype(jnp.int32)
    off_own = jnp.sum(jnp.where(oh_tm, jnp.broadcast_to(off, (S, E)), 0),
                      axis=1, keepdims=True)
    valid = tm >= 0
    pos = off_own + rank_tm - 1

    inv_ref[...] = jnp.where(valid, pos, PADROWS)

    svals = lax.broadcasted_iota(jnp.int32, (S, 1), 0)
    posv = jnp.where(valid, pos, -1)
    for c in range(PADROWS // JC):
        jj = c * JC + lax.broadcasted_iota(jnp.int32, (S, JC), 1)
        hit = posv == jj
        vals = jnp.sum(jnp.where(hit, jnp.broadcast_to(svals, (S, JC)), 0),
                       axis=0, keepdims=True)
        occ = jnp.sum(hit.astype(jnp.int32), axis=0, keepdims=True)
        jrow = c * JC + lax.broadcasted_iota(jnp.int32, (1, JC), 1)
        srow_ref[:, pl.ds(c * JC, JC)] = jnp.where(occ > 0, vals, jrow)

    bj = BLK * lax.broadcasted_iota(jnp.int32, (NB, 1), 0)
    offb = jnp.broadcast_to(off, (NB, E))
    pcb = jnp.broadcast_to(pc, (NB, E))
    inb = (bj >= offb) & (bj < offb + pcb)
    be = jnp.sum(jnp.where(inb, lax.broadcasted_iota(jnp.int32, (NB, E), 1), 0),
                 axis=1, keepdims=True)
    e_last = jnp.max(jnp.where(pc > 0, e_row, 0))
    used = jnp.sum(inb.astype(jnp.int32), axis=1, keepdims=True) > 0
    be_ref[...] = jnp.where(used, be, e_last)

    dens = (hist0 + hist1) * (1.0 / TOTAL)
    probs = probs_sum * (1.0 / TOTAL)
    loss_ref[...] = jnp.sum(probs * dens, axis=1, keepdims=True) * float(E)


def _run_router(xf, Wg, bg2):
    return pl.pallas_call(
        _router_body,
        out_shape=[
            jax.ShapeDtypeStruct((1, PADROWS), jnp.int32),
            jax.ShapeDtypeStruct((S, 1), jnp.int32),
            jax.ShapeDtypeStruct((NB, 1), jnp.int32),
            jax.ShapeDtypeStruct((1, 1), jnp.float32),
        ],
        scratch_shapes=[
            pltpu.VMEM((TOTAL, E), jnp.float32),
            pltpu.VMEM((TOTAL, E), jnp.float32),
        ],
    )(xf, Wg, bg2)


def _ffn_body(be_ref, x_ref, w1_ref, b1_ref, w2_ref, b2_ref, y_ref):
    i = pl.program_id(0)
    h = jnp.maximum(
        jnp.dot(x_ref[...], w1_ref[0], preferred_element_type=jnp.float32)
        + b1_ref[0], 0.0)
    y = jnp.dot(h, w2_ref[0], preferred_element_type=jnp.float32) + b2_ref[0]
    y_ref[...] = jnp.where(i == NB, 0.0, y)


def _run_ffn(be_flat, x_sorted, W1, b1, W2, b2):
    last = NB - 1
    grid_spec = pltpu.PrefetchScalarGridSpec(
        num_scalar_prefetch=1,
        grid=(NB + 1,),
        in_specs=[
            pl.BlockSpec((BLK, D), lambda i, be: (jnp.minimum(i, last), 0)),
            pl.BlockSpec((1, D, D), lambda i, be: (be[jnp.minimum(i, last)], 0, 0)),
            pl.BlockSpec((1, 1, D), lambda i, be: (be[jnp.minimum(i, last)], 0, 0)),
            pl.BlockSpec((1, D, D), lambda i, be: (be[jnp.minimum(i, last)], 0, 0)),
            pl.BlockSpec((1, 1, D), lambda i, be: (be[jnp.minimum(i, last)], 0, 0)),
        ],
        out_specs=pl.BlockSpec((BLK, D), lambda i, be: (i, 0)),
    )
    return pl.pallas_call(
        _ffn_body,
        grid_spec=grid_spec,
        out_shape=jax.ShapeDtypeStruct((YROWS, D), jnp.float32),
    )(be_flat, x_sorted, W1, b1.reshape(E, 1, D), W2, b2.reshape(E, 1, D))


def _make_sc_gather(n_rows_table, n_out, n_chunks):
    b_per_w = n_out // NW
    c = b_per_w // n_chunks
    mesh = plsc.VectorSubcoreMesh(core_axis_name="c", subcore_axis_name="s",
                                  num_cores=NC, num_subcores=NS)

    def body(table_hbm, idx_hbm, out_hbm, idx_v, rows_v, sem):
        wid = lax.axis_index("s") * NC + lax.axis_index("c")
        base = wid * b_per_w
        pltpu.sync_copy(idx_hbm.at[pl.ds(base, b_per_w)], idx_v)
        for k in range(n_chunks):
            off = base + k * c
            pltpu.async_copy(
                table_hbm.at[idx_v.at[pl.ds(k * c, c)]], rows_v, sem).wait()
            pltpu.sync_copy(rows_v, out_hbm.at[pl.ds(off, c)])

    return pl.kernel(
        body,
        out_type=jax.ShapeDtypeStruct((n_out, D), jnp.float32),
        mesh=mesh,
        scratch_types=[
            pltpu.VMEM((b_per_w,), jnp.int32),
            pltpu.VMEM((c, D), jnp.float32),
            pltpu.SemaphoreType.DMA,
        ],
    )


def _make_sc_combine():
    bw = S // NW
    mesh = plsc.VectorSubcoreMesh(core_axis_name="c", subcore_axis_name="s",
                                  num_cores=NC, num_subcores=NS)

    def body(table_hbm, idx_hbm, zeros_hbm, out_hbm, idx_v, rows_v, zbuf,
             gsem, zsem):
        wid = lax.axis_index("s") * NC + lax.axis_index("c")
        base = wid * bw
        zh = pltpu.async_copy(zeros_hbm.at[pl.ds(wid * (bw // 2), bw // 2)],
                              zbuf, zsem)
        pltpu.sync_copy(idx_hbm.at[pl.ds(base, bw)], idx_v)
        pltpu.async_copy(table_hbm.at[idx_v], rows_v, gsem).wait()
        pltpu.sync_copy(rows_v, out_hbm.at[pl.ds(base, bw)])
        zh.wait()
        pltpu.sync_copy(zbuf, out_hbm.at[pl.ds(S + base, bw // 2)])
        pltpu.sync_copy(zbuf, out_hbm.at[pl.ds(S + base + bw // 2, bw // 2)])

    return pl.kernel(
        body,
        out_type=jax.ShapeDtypeStruct((TOTAL, D), jnp.float32),
        mesh=mesh,
        scratch_types=[
            pltpu.VMEM((bw,), jnp.int32),
            pltpu.VMEM((bw, D), jnp.float32),
            pltpu.VMEM((bw // 2, D), jnp.float32),
            pltpu.SemaphoreType.DMA,
            pltpu.SemaphoreType.DMA,
        ],
    )


@jax.jit
def kernel(x, Wg, bg, W1, b1, W2, b2):
    xf = x.reshape(TOTAL, D)
    srow, inv, be, loss = _run_router(xf, Wg, bg.reshape(1, E))
    x_sorted = _make_sc_gather(TOTAL, PADROWS, 1)(xf, srow.reshape(PADROWS))
    y_ext = _run_ffn(be.reshape(NB), x_sorted, W1, b1, W2, b2)
    zeros_half = jnp.zeros((S // 2, D), jnp.float32)
    out = _make_sc_combine()(y_ext, inv.reshape(S), zeros_half)
    return out.reshape(B, S, D), loss[0, 0]


# speedup vs baseline: 1.5394x; 1.0064x over previous
import functools

import jax
import jax.numpy as jnp
from jax import lax
from jax.experimental import pallas as pl
from jax.experimental.pallas import tpu as pltpu
from jax.experimental.pallas import tpu_sc as plsc

B, S, D, E = 2, 2048, 1024, 8
TOTAL = B * S
CAPACITY = int(TOTAL / E * 1.25)
BLK = 128
NB = 24
PADROWS = NB * BLK
YROWS = PADROWS + BLK
CB = 512
JC = 768

NC, NS = 2, 16
NW = NC * NS


def _router_body(x_ref, wg_ref, bg_ref,
                 srow_ref, inv_ref, be_ref, loss_ref,
                 ohf_ref, ranks_ref):
    xx = x_ref[...]
    logits = jnp.dot(xx, wg_ref[...],
                     preferred_element_type=jnp.float32) + bg_ref[...]
    mx = jnp.max(logits, axis=1, keepdims=True)
    ex = jnp.exp(logits - mx)
    sm = ex / jnp.sum(ex, axis=1, keepdims=True)
    probs_sum = jnp.sum(sm, axis=0, keepdims=True)

    eids = lax.broadcasted_iota(jnp.int32, (TOTAL, E), 1)
    idx2d = jnp.min(jnp.where(logits == mx, eids, E), axis=1, keepdims=True)
    oh = idx2d == eids
    ohf = oh.astype(jnp.float32)
    hist0 = jnp.sum(ohf[:S], axis=0, keepdims=True)
    hist1 = jnp.sum(ohf[S:], axis=0, keepdims=True)

    tri = (lax.broadcasted_iota(jnp.int32, (CB, CB), 0) >=
           lax.broadcasted_iota(jnp.int32, (CB, CB), 1)).astype(jnp.float32)
    ohf_ref[...] = ohf

    def _cum_step(i, carry):
        blk = ohf_ref[pl.ds(i * CB, CB), :]
        csum = jnp.dot(tri, blk, preferred_element_type=jnp.float32) + carry
        ranks_ref[pl.ds(i * CB, CB), :] = csum
        return csum[CB - 1:CB, :]

    lax.fori_loop(0, TOTAL // CB, _cum_step, jnp.zeros((1, E), jnp.float32))
    inc = ranks_ref[...]

    rank_own = jnp.sum(inc * ohf, axis=1, keepdims=True)
    kept = rank_own <= float(CAPACITY)
    contrib = jnp.where(kept, idx2d, -1)
    tm = jnp.maximum(contrib[:S], contrib[S:])

    e_row = lax.broadcasted_iota(jnp.int32, (1, E), 1)
    sp0 = jnp.max(jnp.where(hist0 > 0, e_row, -1))
    sp1 = jnp.max(jnp.where((hist1 > 0) & (hist0 < CAPACITY), e_row, -1))
    sidx = lax.broadcasted_iota(jnp.int32, (S, 1), 0)
    tm = jnp.where(sidx == 0, jnp.maximum(tm, sp0), tm)
    tm = jnp.where(sidx == 1, jnp.maximum(tm, sp1), tm)

    oh_tm = tm == lax.broadcasted_iota(jnp.int32, (S, E), 1)
    ohtf = oh_tm.astype(jnp.float32)
    cnt = jnp.sum(ohtf, axis=0, keepdims=True).astype(jnp.int32)
    pc = ((cnt + (BLK - 1)) // BLK) * BLK
    triu = (lax.broadcasted_iota(jnp.int32, (E, E), 0) <
            lax.broadcasted_iota(jnp.int32, (E, E), 1)).astype(jnp.float32)
    off = jnp.dot(pc.astype(jnp.float32), triu,
                  preferred_element_type=jnp.float32).astype(jnp.int32)

    ohf_ref[pl.ds(0, S), :] = ohtf

    def _cum_step2(i, carry):
        blk = ohf_ref[pl.ds(i * CB, CB), :]
        csum = jnp.dot(tri, blk, preferred_element_type=jnp.float32) + carry
        ranks_ref[pl.ds(i * CB, CB), :] = csum
        return csum[CB - 1:CB, :]

    lax.fori_loop(0, S // CB, _cum_step2, jnp.zeros((1, E), jnp.float32))
    rk = ranks_ref[pl.ds(0, S), :]
    rank_tm = jnp.sum(rk * ohtf, axis=1, keepdims=True).astype(jnp.int32)
    off_own = jnp.sum(jnp.where(oh_tm, jnp.broadcast_to(off, (S, E)), 0),
                      axis=1, keepdims=True)
    valid = tm >= 0
    pos = off_own + rank_tm - 1

    inv_ref[...] = jnp.where(valid, pos, PADROWS)

    svals = lax.broadcasted_iota(jnp.int32, (S, 1), 0)
    posv = jnp.where(valid, pos, -1)
    for c in range(PADROWS // JC):
        jj = c * JC + lax.broadcasted_iota(jnp.int32, (S, JC), 1)
        hit = posv == jj
        vals = jnp.sum(jnp.where(hit, jnp.broadcast_to(svals, (S, JC)), 0),
                       axis=0, keepdims=True)
        occ = jnp.sum(hit.astype(jnp.int32), axis=0, keepdims=True)
        jrow = c * JC + lax.broadcasted_iota(jnp.int32, (1, JC), 1)
        srow_ref[:, pl.ds(c * JC, JC)] = jnp.where(occ > 0, vals, jrow)

    bj = BLK * lax.broadcasted_iota(jnp.int32, (NB, 1), 0)
    offb = jnp.broadcast_to(off, (NB, E))
    pcb = jnp.broadcast_to(pc, (NB, E))
    inb = (bj >= offb) & (bj < offb + pcb)
    be = jnp.sum(jnp.where(inb, lax.broadcasted_iota(jnp.int32, (NB, E), 1), 0),
                 axis=1, keepdims=True)
    e_last = jnp.max(jnp.where(pc > 0, e_row, 0))
    used = jnp.sum(inb.astype(jnp.int32), axis=1, keepdims=True) > 0
    be_ref[...] = jnp.where(used, be, e_last)

    dens = (hist0 + hist1) * (1.0 / TOTAL)
    probs = probs_sum * (1.0 / TOTAL)
    loss_ref[...] = jnp.sum(probs * dens, axis=1, keepdims=True) * float(E)


def _run_router(xf, Wg, bg2):
    return pl.pallas_call(
        _router_body,
        out_shape=[
            jax.ShapeDtypeStruct((1, PADROWS), jnp.int32),
            jax.ShapeDtypeStruct((S, 1), jnp.int32),
            jax.ShapeDtypeStruct((NB, 1), jnp.int32),
            jax.ShapeDtypeStruct((1, 1), jnp.float32),
        ],
        scratch_shapes=[
            pltpu.VMEM((TOTAL, E), jnp.float32),
            pltpu.VMEM((TOTAL, E), jnp.float32),
        ],
    )(xf, Wg, bg2)


def _ffn_body(be_ref, x_ref, w1_ref, b1_ref, w2_ref, b2_ref, y_ref):
    i = pl.program_id(0)

    @pl.when(i == NB)
    def _zero_block():
        y_ref[...] = jnp.zeros((BLK, D), jnp.float32)

    @pl.when(i != NB)
    def _compute():
        h = jnp.maximum(
            jnp.dot(x_ref[...], w1_ref[0], preferred_element_type=jnp.float32)
            + b1_ref[0], 0.0)
        y_ref[...] = (jnp.dot(h, w2_ref[0], preferred_element_type=jnp.float32)
                      + b2_ref[0])


def _run_ffn(be_flat, x_sorted, W1, b1, W2, b2):
    last = NB - 1
    grid_spec = pltpu.PrefetchScalarGridSpec(
        num_scalar_prefetch=1,
        grid=(NB + 1,),
        in_specs=[
            pl.BlockSpec((BLK, D), lambda i, be: (jnp.minimum(i, last), 0)),
            pl.BlockSpec((1, D, D), lambda i, be: (be[jnp.minimum(i, last)], 0, 0)),
            pl.BlockSpec((1, 1, D), lambda i, be: (be[jnp.minimum(i, last)], 0, 0)),
            pl.BlockSpec((1, D, D), lambda i, be: (be[jnp.minimum(i, last)], 0, 0)),
            pl.BlockSpec((1, 1, D), lambda i, be: (be[jnp.minimum(i, last)], 0, 0)),
        ],
        out_specs=pl.BlockSpec((BLK, D), lambda i, be: (i, 0)),
    )
    return pl.pallas_call(
        _ffn_body,
        grid_spec=grid_spec,
        out_shape=jax.ShapeDtypeStruct((YROWS, D), jnp.float32),
    )(be_flat, x_sorted, W1, b1.reshape(E, 1, D), W2, b2.reshape(E, 1, D))


def _make_sc_gather(n_rows_table, n_out, n_chunks):
    b_per_w = n_out // NW
    c = b_per_w // n_chunks
    mesh = plsc.VectorSubcoreMesh(core_axis_name="c", subcore_axis_name="s",
                                  num_cores=NC, num_subcores=NS)

    def body(table_hbm, idx_hbm, out_hbm, idx_v, rows_v, sem):
        wid = lax.axis_index("s") * NC + lax.axis_index("c")
        base = wid * b_per_w
        pltpu.sync_copy(idx_hbm.at[pl.ds(base, b_per_w)], idx_v)
        for k in range(n_chunks):
            off = base + k * c
            pltpu.async_copy(
                table_hbm.at[idx_v.at[pl.ds(k * c, c)]], rows_v, sem).wait()
            pltpu.sync_copy(rows_v, out_hbm.at[pl.ds(off, c)])

    return pl.kernel(
        body,
        out_type=jax.ShapeDtypeStruct((n_out, D), jnp.float32),
        mesh=mesh,
        scratch_types=[
            pltpu.VMEM((b_per_w,), jnp.int32),
            pltpu.VMEM((c, D), jnp.float32),
            pltpu.SemaphoreType.DMA,
        ],
    )


def _make_sc_combine():
    bw = S // NW
    mesh = plsc.VectorSubcoreMesh(core_axis_name="c", subcore_axis_name="s",
                                  num_cores=NC, num_subcores=NS)

    def body(table_hbm, idx_hbm, zeros_hbm, out_hbm, idx_v, rows_v, zbuf,
             gsem, zsem):
        wid = lax.axis_index("s") * NC + lax.axis_index("c")
        base = wid * bw
        zh = pltpu.async_copy(zeros_hbm.at[pl.ds(wid * (bw // 2), bw // 2)],
                              zbuf, zsem)
        pltpu.sync_copy(idx_hbm.at[pl.ds(base, bw)], idx_v)
        pltpu.async_copy(table_hbm.at[idx_v], rows_v, gsem).wait()
        pltpu.sync_copy(rows_v, out_hbm.at[pl.ds(base, bw)])
        zh.wait()
        pltpu.sync_copy(zbuf, out_hbm.at[pl.ds(S + base, bw // 2)])
        pltpu.sync_copy(zbuf, out_hbm.at[pl.ds(S + base + bw // 2, bw // 2)])

    return pl.kernel(
        body,
        out_type=jax.ShapeDtypeStruct((TOTAL, D), jnp.float32),
        mesh=mesh,
        scratch_types=[
            pltpu.VMEM((bw,), jnp.int32),
            pltpu.VMEM((bw, D), jnp.float32),
            pltpu.VMEM((bw // 2, D), jnp.float32),
            pltpu.SemaphoreType.DMA,
            pltpu.SemaphoreType.DMA,
        ],
    )


@jax.jit
def kernel(x, Wg, bg, W1, b1, W2, b2):
    xf = x.reshape(TOTAL, D)
    srow, inv, be, loss = _run_router(xf, Wg, bg.reshape(1, E))
    x_sorted = _make_sc_gather(TOTAL, PADROWS, 1)(xf, srow.reshape(PADROWS))
    y_ext = _run_ffn(be.reshape(NB), x_sorted, W1, b1, W2, b2)
    zeros_half = jnp.zeros((S // 2, D), jnp.float32)
    out = _make_sc_combine()(y_ext, inv.reshape(S), zeros_half)
    return out.reshape(B, S, D), loss.reshape(())
